# baseline (speedup 1.0000x reference)
import kernel_split as KS
import kernel_r3 as KR

import jax.numpy as jnp


def kernel(x, edge_index, batch, W, b):
    batch2 = batch.reshape(KS.N_NODES // KS.CH, 1, KS.CH)
    x3 = x.reshape(KS.N_NODES // KS.CH, KS.CH, KS.IN_DIM)
    zvec = jnp.zeros((KS.N_GRAPHS, KS.IN_DIM), jnp.float32)
    partials = KS._sc_segsum(x3, batch2, zvec)
    out = KR.kernel(x, edge_index, batch, W, b)
    return out + 0.0 * partials[0, :, :10]
